# Initial kernel scaffold; baseline (speedup 1.0000x reference)
import jax
import jax.numpy as jnp
from jax.experimental import pallas as pl


def kernel(x, src_uid, src_iid, tgt_iid, ek_w1, ek_b1, ek_w2, dec_w1, dec_b1, dec_w2, dec_b2):
    raise NotImplementedError("write your pallas kernel here")


# speedup vs baseline: 3.0911x; 3.0911x over previous
import functools

import jax
import jax.numpy as jnp
from jax import lax
from jax.experimental import pallas as pl
from jax.experimental.pallas import tpu as pltpu
from jax.experimental.pallas import tpu_sc as plsc

_B = 4096
_H = 200
_E = 128
_MF = 128
_BLK = 64
_NW = 32
_ROWS_W = _B * _H // _NW
_CH = 128
_NCH = _ROWS_W // _CH
_BW = _B // _NW


def _sc_gather(src_iid, tgt_iid, src_uid, seq3d, iid2d, uid2d):
    mesh = plsc.VectorSubcoreMesh(core_axis_name="c", subcore_axis_name="s")

    @functools.partial(
        pl.kernel,
        out_type=(
            jax.ShapeDtypeStruct((_B * _H, _E), jnp.float32),
            jax.ShapeDtypeStruct((_B, _E), jnp.float32),
            jax.ShapeDtypeStruct((_B, _E), jnp.float32),
        ),
        mesh=mesh,
        scratch_types=[
            pltpu.VMEM((_NCH, _CH), jnp.int32),
            pltpu.VMEM((_CH, _E), jnp.float32),
            pltpu.VMEM((_CH, _E), jnp.float32),
            pltpu.VMEM((_BW,), jnp.int32),
            pltpu.SemaphoreType.DMA,
            pltpu.SemaphoreType.DMA,
        ],
    )
    def k(src_iid_hbm, tgt_iid_hbm, src_uid_hbm, seq_hbm, iid_hbm, uid_hbm,
          ufea_hbm, v_hbm, u_hbm, idx_v, rows_a, rows_b, sidx, sem_g, sem_s):
        wid = lax.axis_index("s") * 2 + lax.axis_index("c")
        base = wid * _ROWS_W

        pltpu.sync_copy(iid_hbm.at[wid], sidx)
        pltpu.async_copy(tgt_iid_hbm.at[sidx], rows_a, sem_g).wait()
        pltpu.sync_copy(rows_a, v_hbm.at[pl.ds(wid * _BW, _BW)])
        pltpu.sync_copy(uid_hbm.at[wid], sidx)
        pltpu.async_copy(src_uid_hbm.at[sidx], rows_a, sem_g).wait()
        pltpu.sync_copy(rows_a, u_hbm.at[pl.ds(wid * _BW, _BW)])

        pltpu.sync_copy(seq_hbm.at[wid], idx_v)
        pltpu.async_copy(src_iid_hbm.at[idx_v.at[0]], rows_a, sem_g)

        def step2(j2, _):
            j = j2 * 2
            pltpu.make_async_copy(src_iid_hbm.at[idx_v.at[0]], rows_a, sem_g).wait()
            pltpu.async_copy(src_iid_hbm.at[idx_v.at[j + 1]], rows_b, sem_g)
            pltpu.sync_copy(rows_a, ufea_hbm.at[pl.ds(base + j * _CH, _CH)])
            pltpu.make_async_copy(src_iid_hbm.at[idx_v.at[0]], rows_b, sem_g).wait()

            @pl.when(j2 + 1 < _NCH // 2)
            def _():
                pltpu.async_copy(src_iid_hbm.at[idx_v.at[j + 2]], rows_a, sem_g)

            pltpu.sync_copy(rows_b, ufea_hbm.at[pl.ds(base + (j + 1) * _CH, _CH)])
            return 0

        lax.fori_loop(0, _NCH // 2, step2, 0)

    return k(src_iid, tgt_iid, src_uid, seq3d, iid2d, uid2d)


def _tc_compute(ufea, seq3, u, v, w1, b1r, w2r, dw1, db1r, dw2, db2r,
                interpret=False):
    grid = (_B // _BLK,)

    def body(uf_ref, seq_ref, u_ref, v_ref, w1_ref, b1_ref, w2_ref,
             dw1_ref, db1_ref, dw2_ref, db2_ref, out_ref):
        uf3 = uf_ref[...]
        uf2 = uf3.reshape(_BLK * _H, _E)
        h = jnp.maximum(
            jnp.dot(uf2, w1_ref[...], preferred_element_type=jnp.float32)
            + b1_ref[...], 0.0)
        ek = jnp.sum(h * w2_ref[...], axis=1, keepdims=True)
        ek3 = ek.reshape(_BLK, _H, 1)
        m3 = (seq_ref[...] == 0).astype(jnp.float32)
        t = ek3 - m3 * 1e8
        mx = jnp.max(t, axis=1, keepdims=True)
        p = jnp.exp(t - mx)
        att = p / jnp.sum(p, axis=1, keepdims=True)
        his = jnp.sum(att * uf3, axis=1)
        g = jnp.maximum(
            jnp.dot(his, dw1_ref[...], preferred_element_type=jnp.float32)
            + db1_ref[...], 0.0)
        z = jnp.dot(g, dw2_ref[...], preferred_element_type=jnp.float32) \
            + db2_ref[...]
        z3 = z.reshape(_BLK, _E, _E)
        w = jnp.sum(z3 * v_ref[...][:, None, :], axis=2)
        out_ref[...] = jnp.sum(w * u_ref[...], axis=1, keepdims=True)

    return pl.pallas_call(
        body,
        grid=grid,
        in_specs=[
            pl.BlockSpec((_BLK, _H, _E), lambda i: (i, 0, 0)),
            pl.BlockSpec((_BLK, _H, 1), lambda i: (i, 0, 0)),
            pl.BlockSpec((_BLK, _E), lambda i: (i, 0)),
            pl.BlockSpec((_BLK, _E), lambda i: (i, 0)),
            pl.BlockSpec((_E, _E), lambda i: (0, 0)),
            pl.BlockSpec((1, _E), lambda i: (0, 0)),
            pl.BlockSpec((1, _E), lambda i: (0, 0)),
            pl.BlockSpec((_E, _MF), lambda i: (0, 0)),
            pl.BlockSpec((1, _MF), lambda i: (0, 0)),
            pl.BlockSpec((_MF, _E * _E), lambda i: (0, 0)),
            pl.BlockSpec((1, _E * _E), lambda i: (0, 0)),
        ],
        out_specs=pl.BlockSpec((_BLK, 1), lambda i: (i, 0)),
        out_shape=jax.ShapeDtypeStruct((_B, 1), jnp.float32),
        interpret=interpret,
    )(ufea, seq3, u, v, w1, b1r, w2r, dw1, db1r, dw2, db2r)


def kernel(x, src_uid, src_iid, tgt_iid, ek_w1, ek_b1, ek_w2,
           dec_w1, dec_b1, dec_w2, dec_b2):
    x = x.astype(jnp.int32)
    seq = x[:, 2:]
    seq3d = seq.reshape(_NW, _NCH, _CH)
    iid2d = x[:, 1].reshape(_NW, _BW)
    uid2d = x[:, 0].reshape(_NW, _BW)

    ufea_rows, v, u = _sc_gather(src_iid, tgt_iid, src_uid,
                                 seq3d, iid2d, uid2d)
    ufea = ufea_rows.reshape(_B, _H, _E)
    seq3 = seq.reshape(_B, _H, 1)

    out2 = _tc_compute(
        ufea, seq3, u, v,
        ek_w1, ek_b1.reshape(1, _E), ek_w2.reshape(1, _E),
        dec_w1, dec_b1.reshape(1, _MF), dec_w2, dec_b2.reshape(1, _E * _E))
    return out2[:, 0]
